# Initial kernel scaffold; baseline (speedup 1.0000x reference)
import jax
import jax.numpy as jnp
from jax.experimental import pallas as pl


def kernel(x, img_size, W_share, b_share, W_cls, b_cls, W_reg, b_reg):
    raise NotImplementedError("write your pallas kernel here")


# speedup vs baseline: 1.0063x; 1.0063x over previous
import jax
import jax.numpy as jnp
import numpy as np
from jax import lax
from jax.experimental import pallas as pl


def _anchor_base_np(base_size=16.0, ratios=(0.5, 1.0, 2.0), scales=(8.0, 16.0, 32.0)):
    ctr = base_size / 2.0
    ab = []
    for r in ratios:
        for s in scales:
            h = base_size * s * np.sqrt(r)
            w = base_size * s * np.sqrt(1.0 / r)
            ab.append([ctr - w / 2.0, ctr - h / 2.0, ctr + w / 2.0, ctr + h / 2.0])
    return np.array(ab, dtype=np.float32)


def _all_anchors_np(stride, h, w):
    ab = _anchor_base_np()
    shift_x = np.arange(w, dtype=np.float32) * stride
    shift_y = np.arange(h, dtype=np.float32) * stride
    sx, sy = np.meshgrid(shift_x, shift_y)
    shifts = np.stack([sx.ravel(), sy.ravel(), sx.ravel(), sy.ravel()], axis=1)
    anchors = shifts[:, None, :] + ab[None, :, :]
    return anchors.reshape(-1, 4)


def _decode_kernel(a_ref, l_ref, img_ref, o_ref):
    a = a_ref[...]
    l = l_ref[...]
    img = img_ref[0]
    aw = a[2:3, :] - a[0:1, :]
    ah = a[3:4, :] - a[1:2, :]
    acx = a[0:1, :] + 0.5 * aw
    acy = a[1:2, :] + 0.5 * ah
    cx = acx + l[0:1, :] * aw
    cy = acy + l[1:2, :] * ah
    w = aw * jnp.exp(jnp.clip(l[2:3, :], -10.0, 10.0))
    h = ah * jnp.exp(jnp.clip(l[3:4, :], -10.0, 10.0))
    rois = jnp.concatenate([cx - 0.5 * w, cy - 0.5 * h, cx + 0.5 * w, cy + 0.5 * h], axis=0)
    o_ref[...] = jnp.clip(rois, 0.0, img)


def _decode(anchors, loc, img):
    out = pl.pallas_call(
        _decode_kernel,
        out_shape=jax.ShapeDtypeStruct((4, anchors.shape[0]), jnp.float32),
    )(anchors.T, loc.T, jnp.full((1,), img, dtype=jnp.float32))
    return out.T


def _conv(x, W, b, pad):
    y = lax.conv_general_dilated(x, W, (1, 1), pad, dimension_numbers=('NCHW', 'OIHW', 'NCHW'))
    return y + b[None, :, None, None]


def _nms_keep(boxes, thresh):
    n = boxes.shape[0]
    x1, y1, x2, y2 = boxes[:, 0], boxes[:, 1], boxes[:, 2], boxes[:, 3]
    area = jnp.maximum(x2 - x1, 0.0) * jnp.maximum(y2 - y1, 0.0)
    idxs = jnp.arange(n)

    def body(i, keep):
        xx1 = jnp.maximum(x1[i], x1)
        yy1 = jnp.maximum(y1[i], y1)
        xx2 = jnp.minimum(x2[i], x2)
        yy2 = jnp.minimum(y2[i], y2)
        inter = jnp.maximum(xx2 - xx1, 0.0) * jnp.maximum(yy2 - yy1, 0.0)
        iou = inter / (area[i] + area - inter + 1e-9)
        suppress = (iou > thresh) & (idxs > i) & keep[i]
        return keep & (~suppress)

    return lax.fori_loop(0, n, body, jnp.ones((n,), dtype=bool))


def _proposal(score, rois, img, n_pre=6000, n_post=300, nms_thresh=0.7, min_size=16.0):
    ws = rois[:, 2] - rois[:, 0]
    hs = rois[:, 3] - rois[:, 1]
    valid = (ws >= min_size) & (hs >= min_size)
    score = jnp.where(valid, score, -jnp.inf)
    _, order = lax.top_k(score, n_pre)
    boxes = rois[order]
    keep = _nms_keep(boxes, nms_thresh)
    sel = jnp.where(keep, size=n_post, fill_value=0)[0]
    return boxes[sel]


def kernel(x, img_size, W_share, b_share, W_cls, b_cls, W_reg, b_reg):
    bat, _, h, w = x.shape
    anchors = jnp.asarray(_all_anchors_np(16.0, h, w))
    shared = jax.nn.relu(_conv(x, W_share, b_share, 'SAME'))
    cls = _conv(shared, W_cls, b_cls, 'VALID')
    cls = jnp.transpose(cls, (0, 2, 3, 1)).reshape(bat, h, w, 9, 2)
    cls = jax.nn.softmax(cls, axis=4).reshape(bat, -1, 2)
    reg = _conv(shared, W_reg, b_reg, 'VALID')
    reg = jnp.transpose(reg, (0, 2, 3, 1)).reshape(bat, -1, 4)

    img = jnp.asarray(img_size, dtype=jnp.float32)
    roi_list = []
    roi_ids = []
    for i in range(bat):
        rois_i = _decode(anchors, reg[i], img)
        roi = _proposal(cls[i][:, 1], rois_i, img)
        roi_list.append(roi)
        roi_ids.append(jnp.full((roi.shape[0],), i, dtype=jnp.int32))
    rois = jnp.concatenate(roi_list, axis=0)
    roi_id = jnp.concatenate(roi_ids, axis=0)
    return (reg, cls, rois, roi_id, anchors)
